# Initial kernel scaffold; baseline (speedup 1.0000x reference)
import jax
import jax.numpy as jnp
from jax.experimental import pallas as pl


def kernel(feats, programs, stem_w1, stem_b1, stem_w2, stem_b2, uw1, ub1, uw2, ub2, bwp, bbp, bw1, bb1, bw2, bb2, cls_w, cls_b, fc1_w, fc1_b, fc2_w, fc2_b):
    raise NotImplementedError("write your pallas kernel here")


# speedup vs baseline: 5.7117x; 5.7117x over previous
import functools

import jax
import jax.numpy as jnp
from jax.experimental import pallas as pl
from jax.experimental.pallas import tpu as pltpu

_BRANCH = jnp.array([0, 0, 0, 0, 1, 2, 2, 2, 2, 2, 2, 2, 2, 2, 2, 3, 3, 3, 3, 3, 3],
                    dtype=jnp.int32)
_PIDX = jnp.array([0, 0, 0, 0, 0, 1, 2, 3, 4, 5, 6, 7, 8, 9, 10, 0, 1, 2, 3, 4, 5],
                  dtype=jnp.int32)
_BR3 = jnp.array([0, 1, 1, 2], dtype=jnp.int32)

_L = 25
_S = 256
_D = 128
_SHIFTS = [-17, -16, -15, -1, 0, 1, 15, 16, 17]

_F32 = jnp.float32


def _roll_lanes(y, m):
    m = m % _S
    if m == 0:
        return y
    return jnp.concatenate([y[:, _S - m:], y[:, :_S - m]], axis=1)


def _conv3(x, bank, pi):
    acc = jnp.dot(bank[pi, 4], x, preferred_element_type=_F32)
    for k in range(9):
        if k == 4:
            continue
        y = jnp.dot(bank[pi, k], x, preferred_element_type=_F32)
        acc = acc + _roll_lanes(y, -_SHIFTS[k])
    return acc


def _fused_kernel(code_ref, xpad_ref, w1t_ref, w2t_ref, uw1t_ref, uw2t_ref,
                  wpa_ref, wpb_ref, bw1t_ref, bw2t_ref, clsw_ref,
                  b1b_ref, b2b_ref, ub1b_ref, ub2b_ref, bbpb_ref, bb1b_ref,
                  bb2b_ref, clsbb_ref, out_ref, feat_ref):
    b = pl.program_id(0)
    relu = jax.nn.relu

    lane = jax.lax.broadcasted_iota(jnp.int32, (_D, _S), 1)
    r = lane >> 4
    c = lane & 15
    mask = jnp.where((r >= 1) & (r <= 14) & (c >= 1) & (c <= 14),
                     jnp.float32(1.0), jnp.float32(0.0))

    x = xpad_ref[0]
    acc = jnp.dot(w1t_ref[4], x, preferred_element_type=_F32)
    for k in range(9):
        if k == 4:
            continue
        y = jnp.dot(w1t_ref[k], x, preferred_element_type=_F32)
        acc = acc + _roll_lanes(y, -_SHIFTS[k])
    v0 = relu(acc + b1b_ref[...]) * mask

    a2 = _conv3(v0, w2t_ref, 0)
    feat = relu(a2 + b2b_ref[...]) * mask
    feat_ref[...] = feat

    def step(l, carry):
        out, sv = carry
        code = code_ref[b, l]
        br3 = code >> 6
        sc = ((code >> 5) & 1) == 1
        pi = code & 31

        def arm_noop(o, s):
            return o, s

        def arm_unary(o, s):
            ft = feat_ref[...]
            xin = jnp.where(sc, ft, o)
            s2 = jnp.where(sc, o, s)
            h = relu(_conv3(xin, uw1t_ref, pi) + ub1b_ref[pi]) * mask
            h2 = _conv3(h, uw2t_ref, pi) + ub2b_ref[pi]
            return relu(xin + h2) * mask, s2

        def arm_binary(o, s):
            y0 = (jnp.dot(wpa_ref[pi], o, preferred_element_type=_F32)
                  + jnp.dot(wpb_ref[pi], s, preferred_element_type=_F32)
                  + bbpb_ref[pi])
            y = relu(y0) * mask
            h = relu(_conv3(y, bw1t_ref, pi) + bb1b_ref[pi]) * mask
            h2 = _conv3(h, bw2t_ref, pi) + bb2b_ref[pi]
            return relu(y + h2) * mask, s

        return jax.lax.switch(br3, (arm_noop, arm_unary, arm_binary), out, sv)

    out, _ = jax.lax.fori_loop(0, _L, step, (feat, feat))

    ct = relu(jnp.dot(clsw_ref[...], out, preferred_element_type=_F32)
              + clsbb_ref[...])
    e = jnp.maximum(ct, _roll_lanes(ct, -1))
    f = jnp.maximum(e, _roll_lanes(e, -16))
    out_ref[0] = f


def _fc1_kernel(x_ref, w_ref, o_ref, acc_ref):
    ki = pl.program_id(1)

    @pl.when(ki == 0)
    def _():
        acc_ref[...] = jnp.zeros_like(acc_ref)

    acc_ref[...] += jax.lax.dot_general(
        x_ref[...], w_ref[...], (((1,), (1,)), ((), ())),
        preferred_element_type=_F32)

    @pl.when(ki == pl.num_programs(1) - 1)
    def _():
        o_ref[...] = acc_ref[...]


def _fc2_kernel(h_ref, b1_ref, w2_ref, b2_ref, o_ref):
    hh = jax.nn.relu(h_ref[...] + b1_ref[...])
    o_ref[...] = jax.lax.dot_general(
        hh, w2_ref[...], (((1,), (1,)), ((), ())),
        preferred_element_type=_F32) + b2_ref[...]


def kernel(feats, programs, stem_w1, stem_b1, stem_w2, stem_b2,
           uw1, ub1, uw2, ub2, bwp, bbp, bw1, bb1, bw2, bb2,
           cls_w, cls_b, fc1_w, fc1_b, fc2_w, fc2_b):
    B = feats.shape[0]

    xpad = jnp.pad(feats, ((0, 0), (0, 0), (1, 1), (1, 1))).reshape(B, 1024, _S)

    progs_r = programs[:, ::-1].astype(jnp.int32)
    br = _BRANCH[progs_r]
    pi = _PIDX[progs_r]
    code = (_BR3[br] << 6) | (jnp.where(br == 1, 1, 0) << 5) | pi
    code = code.astype(jnp.int32)

    def tapmat(w):
        return jnp.moveaxis(w, (-2, -1), (-4, -3)).reshape(
            w.shape[:-4] + (9, w.shape[-4], w.shape[-3]))

    w1t = tapmat(stem_w1)
    w2t = tapmat(stem_w2)[None]
    uw1t = tapmat(uw1)
    uw2t = tapmat(uw2)
    bw1t = tapmat(bw1)
    bw2t = tapmat(bw2)
    wpa = bwp[:, :, :_D, 0, 0]
    wpb = bwp[:, :, _D:, 0, 0]
    clsw = cls_w[:, :, 0, 0]

    bcast = lambda v: jnp.broadcast_to(v[..., None], v.shape + (_S,))
    b1b = bcast(stem_b1)
    b2b = bcast(stem_b2)
    ub1b = bcast(ub1)
    ub2b = bcast(ub2)
    bbpb = bcast(bbp)
    bb1b = bcast(bb1)
    bb2b = bcast(bb2)
    clsbb = bcast(cls_b)

    full = lambda shape: pl.BlockSpec(shape, lambda b, code: (0,) * len(shape))

    pooled_t = pl.pallas_call(
        _fused_kernel,
        grid_spec=pltpu.PrefetchScalarGridSpec(
            num_scalar_prefetch=1,
            grid=(B,),
            in_specs=[
                pl.BlockSpec((1, 1024, _S), lambda b, code: (b, 0, 0)),
                full((9, _D, 1024)),
                full((1, 9, _D, _D)),
                full((11, 9, _D, _D)),
                full((11, 9, _D, _D)),
                full((6, _D, _D)),
                full((6, _D, _D)),
                full((6, 9, _D, _D)),
                full((6, 9, _D, _D)),
                full((512, _D)),
                full((_D, _S)),
                full((_D, _S)),
                full((11, _D, _S)),
                full((11, _D, _S)),
                full((6, _D, _S)),
                full((6, _D, _S)),
                full((6, _D, _S)),
                full((512, _S)),
            ],
            out_specs=pl.BlockSpec((1, 512, _S), lambda b, code: (b, 0, 0)),
            scratch_shapes=[pltpu.VMEM((_D, _S), _F32)],
        ),
        out_shape=jax.ShapeDtypeStruct((B, 512, _S), _F32),
        compiler_params=pltpu.CompilerParams(
            dimension_semantics=("arbitrary",),
            vmem_limit_bytes=56 * 1024 * 1024,
        ),
    )(code, xpad, w1t, w2t, uw1t, uw2t, wpa, wpb, bw1t, bw2t, clsw,
      b1b, b2b, ub1b, ub2b, bbpb, bb1b, bb2b, clsbb)

    flat = pooled_t.reshape(B, 512, 16, 16)[:, :, 1:15:2, 1:15:2].reshape(B, 512 * 49)

    KC = 6272
    h = pl.pallas_call(
        _fc1_kernel,
        grid=(2, 4),
        in_specs=[
            pl.BlockSpec((B, KC), lambda n, k: (0, k)),
            pl.BlockSpec((512, KC), lambda n, k: (n, k)),
        ],
        out_specs=pl.BlockSpec((B, 512), lambda n, k: (0, n)),
        scratch_shapes=[pltpu.VMEM((B, 512), _F32)],
        out_shape=jax.ShapeDtypeStruct((B, 1024), _F32),
        compiler_params=pltpu.CompilerParams(
            dimension_semantics=("parallel", "arbitrary"),
            vmem_limit_bytes=56 * 1024 * 1024,
        ),
    )(flat, fc1_w)

    out = pl.pallas_call(
        _fc2_kernel,
        in_specs=[
            pl.BlockSpec((B, 1024), lambda: (0, 0)),
            pl.BlockSpec((1, 1024), lambda: (0, 0)),
            pl.BlockSpec((32, 1024), lambda: (0, 0)),
            pl.BlockSpec((1, 32), lambda: (0, 0)),
        ],
        out_specs=pl.BlockSpec((B, 32), lambda: (0, 0)),
        out_shape=jax.ShapeDtypeStruct((B, 32), _F32),
        compiler_params=pltpu.CompilerParams(
            vmem_limit_bytes=56 * 1024 * 1024,
        ),
    )(h, fc1_b.reshape(1, 1024), fc2_w, fc2_b.reshape(1, 32))

    return out
